# Initial kernel scaffold; baseline (speedup 1.0000x reference)
import jax
import jax.numpy as jnp
from jax.experimental import pallas as pl


def kernel(x, W_rel, b_rel, W_self, b_self, edge_index, edge_type, node_type):
    raise NotImplementedError("write your pallas kernel here")


# speedup vs baseline: 13.2561x; 13.2561x over previous
import functools

import jax
import jax.numpy as jnp
from jax import lax
from jax.experimental import pallas as pl
from jax.experimental.pallas import tpu as pltpu
from jax.experimental.pallas import tpu_sc as plsc

_NC = 2
_NS = 16
_NW = _NC * _NS
_CH = 128


def _transform_body(x_ref, w_ref, b_ref, out_ref):
    out_ref[0] = (
        jnp.dot(x_ref[...], w_ref[0], preferred_element_type=jnp.float32) + b_ref[0]
    )


def _rel_transform(x, W_rel, b_rel, nb):
    N, D = x.shape
    R = W_rel.shape[0]
    return pl.pallas_call(
        _transform_body,
        grid=(N // nb, R),
        in_specs=[
            pl.BlockSpec((nb, D), lambda i, r: (i, 0)),
            pl.BlockSpec((1, D, D), lambda i, r: (r, 0, 0)),
            pl.BlockSpec((1, 1, D), lambda i, r: (r, 0, 0)),
        ],
        out_specs=pl.BlockSpec((1, nb, D), lambda i, r: (r, i, 0)),
        out_shape=jax.ShapeDtypeStruct((R, N, D), jnp.float32),
    )(x, W_rel, b_rel.reshape(R, 1, D))


def _make_sc_agg(RN, D, K, NP):
    stripe = NP // _NS
    mesh = plsc.VectorSubcoreMesh(
        core_axis_name="c", subcore_axis_name="s", num_cores=_NC, num_subcores=_NS)

    @functools.partial(
        pl.kernel,
        out_type=[
            jax.ShapeDtypeStruct((_NC, NP, D), jnp.float32),
            jax.ShapeDtypeStruct((_NW, NP), jnp.float32),
        ],
        mesh=mesh,
        compiler_params=pltpu.CompilerParams(needs_layout_passes=False),
        scratch_types=[
            pltpu.VMEM((_CH,), jnp.int32),
            pltpu.VMEM((_CH,), jnp.int32),
            pltpu.VMEM((_CH,), jnp.int32),
            pltpu.VMEM((_CH, D), jnp.float32),
            pltpu.VMEM((NP,), jnp.float32),
            pltpu.VMEM_SHARED((NP, D), jnp.float32),
            pltpu.SemaphoreType.DMA,
        ],
    )
    def sc_agg(table, pkidx, zrows, zdeg, accp, degp,
               pk_c, src_c, dst_c, rows_v, deg_v, acc_s, sem):
        c = lax.axis_index("c")
        s = lax.axis_index("s")
        wid = s * _NC + c
        pltpu.sync_copy(zdeg, deg_v)
        row0 = s * stripe
        for z in range(stripe // _CH):
            pltpu.sync_copy(zrows, acc_s.at[pl.ds(row0 + z * _CH, _CH)])
        plsc.subcore_barrier()
        ones16 = jnp.ones((16,), jnp.float32)

        def body(k, carry):
            pltpu.sync_copy(pkidx.at[wid, k], pk_c)
            for j in range(_CH // 16):
                v = pk_c[pl.ds(j * 16, 16)]
                src_c[pl.ds(j * 16, 16)] = lax.shift_right_logical(v, 14)
                dstv = lax.bitwise_and(v, 16383)
                dst_c[pl.ds(j * 16, 16)] = dstv
                plsc.addupdate_scatter(deg_v, [dstv], ones16)
            pltpu.async_copy(table.at[src_c], rows_v, sem).wait()
            pltpu.sync_copy(rows_v, acc_s.at[dst_c], add=True)
            return carry

        lax.fori_loop(0, K, body, 0)
        plsc.subcore_barrier()
        pltpu.sync_copy(acc_s.at[pl.ds(row0, stripe)], accp.at[c, pl.ds(row0, stripe)])
        pltpu.sync_copy(deg_v, degp.at[wid])

    return sc_agg


def _combine_body(T, accp_ref, deg_ref, x_ref, nt_ref, ws_ref, bs_ref, out_ref):
    nb = out_ref.shape[0]
    rows = nb // 128
    acc = accp_ref[0] + accp_ref[1]
    degm = jnp.sum(deg_ref[...], axis=0)
    ir = lax.broadcasted_iota(jnp.int32, (nb, rows), 0)
    ic = lax.broadcasted_iota(jnp.int32, (nb, rows), 1)
    P = (ir // 128 == ic).astype(jnp.float32)
    C = jnp.dot(P, degm, preferred_element_type=jnp.float32)
    jr = lax.broadcasted_iota(jnp.int32, (nb, 128), 0)
    jc = lax.broadcasted_iota(jnp.int32, (nb, 128), 1)
    deg_n = jnp.sum(jnp.where(jr % 128 == jc, C, 0.0), axis=1, keepdims=True)
    h = acc / jnp.maximum(deg_n, 1.0)
    xb = x_ref[...]
    nt = nt_ref[0, 0, :].reshape(nb, 1)
    for t in range(T):
        ht = jnp.dot(xb, ws_ref[t], preferred_element_type=jnp.float32) + bs_ref[t]
        h = h + jnp.where(nt == t, 1.0, 0.0) * ht
    out_ref[...] = jnp.where(h > 0, h, jnp.exp(h) - 1.0)


def _combine(accp, deg_stack, x_pad, nt_pad, W_self, b_self, NP, nb):
    T, D, _ = W_self.shape
    rows = nb // 128
    return pl.pallas_call(
        functools.partial(_combine_body, T),
        grid=(NP // nb,),
        in_specs=[
            pl.BlockSpec((_NC, nb, D), lambda i: (0, i, 0)),
            pl.BlockSpec((_NW, rows, 128), lambda i: (0, i, 0)),
            pl.BlockSpec((nb, D), lambda i: (i, 0)),
            pl.BlockSpec((1, 1, nb), lambda i: (i, 0, 0)),
            pl.BlockSpec((T, D, D), lambda i: (0, 0, 0)),
            pl.BlockSpec((T, 1, D), lambda i: (0, 0, 0)),
        ],
        out_specs=pl.BlockSpec((nb, D), lambda i: (i, 0)),
        out_shape=jax.ShapeDtypeStruct((NP, D), jnp.float32),
    )(accp, deg_stack, x_pad, nt_pad, W_self, b_self.reshape(T, 1, D))


def kernel(x, W_rel, b_rel, W_self, b_self, edge_index, edge_type, node_type):
    N, D = x.shape
    R = W_rel.shape[0]
    T = W_self.shape[0]
    E = edge_index.shape[1]

    NBC = 2048
    K = -(-E // (_NW * _CH))
    EP = _NW * K * _CH
    NP = (N // NBC + 1) * NBC

    src = edge_index[0]
    dst = edge_index[1]
    flat_src = edge_type * N + src
    pad = EP - E
    flat_src = jnp.pad(flat_src, (0, pad))
    dst_p = jnp.pad(dst, (0, pad), constant_values=NP - 1)
    packed = (flat_src * 16384 + dst_p).reshape(_NW, K, _CH)

    table = _rel_transform(x, W_rel, b_rel, 2000).reshape(R * N, D)

    zrows = jnp.zeros((_CH, D), jnp.float32)
    zdeg = jnp.zeros((NP,), jnp.float32)
    accp, degp = _make_sc_agg(R * N, D, K, NP)(table, packed, zrows, zdeg)

    deg_stack = degp.reshape(_NW, NP // 128, 128)
    x_pad = jnp.pad(x, ((0, NP - N), (0, 0)))
    nt_pad = jnp.pad(node_type, (0, NP - N)).reshape(NP // NBC, 1, NBC)

    out = _combine(accp, deg_stack, x_pad, nt_pad, W_self, b_self, NP, NBC)
    return out[:N]
